# Initial kernel scaffold; baseline (speedup 1.0000x reference)
import functools

import jax
import jax.numpy as jnp
import numpy as np
from jax import lax
from jax.experimental import pallas as pl
from jax.experimental.pallas import tpu as pltpu
from jax.experimental.pallas import tpu_sc as plsc

B = 4096
F = 26
D = 16
FIELD = 3846
V = F * FIELD
H1, H2 = 256, 128
EOD = F * D

NPAIR = (F * (F - 1)) // 2
PAIR_PAD = 328

_iu, _ju = np.triu_indices(F, 1)
_colP = np.zeros(PAIR_PAD, np.int32)
_colP[:NPAIR] = _iu
_tabP = np.zeros(PAIR_PAD, np.int32)
_tabP[:NPAIR] = _ju
_colQ = np.zeros(PAIR_PAD, np.int32)
_colQ[:NPAIR] = _ju
_tabQ = np.zeros(PAIR_PAD, np.int32)
_tabQ[:NPAIR] = _iu

NW = 32
SPW = B // NW
S = 4
NCHUNK = SPW // S
ROWS = S * PAIR_PAD
NFULL = ROWS // 128
REM = ROWS - NFULL * 128
EROWS = S * F


def _sc_body(wffm, wemb, wlin, idxp, idxq, xif,
             ffm_out, emb_out, lin_out,
             idxp_v, idxq_v, xif_v, p_v, q_v, e_v, l_v, acc_v, sem):
  cid = lax.axis_index("c")
  sid = lax.axis_index("s")
  wid = sid * 2 + cid

  def chunk(c, carry):
    base_s = wid * SPW + c * S
    fp = base_s * PAIR_PAD
    fe = base_s * F
    pltpu.sync_copy(idxp.at[pl.ds(fp, ROWS)], idxp_v)
    pltpu.sync_copy(idxq.at[pl.ds(fp, ROWS)], idxq_v)
    pltpu.sync_copy(xif.at[pl.ds(fe, EROWS)], xif_v)
    copies = []
    for k in range(NFULL):
      copies.append(pltpu.async_copy(
          wffm.at[idxp_v.at[pl.ds(k * 128, 128)]],
          p_v.at[pl.ds(k * 128, 128)], sem))
      copies.append(pltpu.async_copy(
          wffm.at[idxq_v.at[pl.ds(k * 128, 128)]],
          q_v.at[pl.ds(k * 128, 128)], sem))
    copies.append(pltpu.async_copy(
        wffm.at[idxp_v.at[pl.ds(NFULL * 128, REM)]],
        p_v.at[pl.ds(NFULL * 128, REM)], sem))
    copies.append(pltpu.async_copy(
        wffm.at[idxq_v.at[pl.ds(NFULL * 128, REM)]],
        q_v.at[pl.ds(NFULL * 128, REM)], sem))
    copies.append(pltpu.async_copy(wemb.at[xif_v], e_v, sem))
    copies.append(pltpu.async_copy(wlin.at[xif_v], l_v, sem))
    for cp in copies:
      cp.wait()
    for s in range(S):
      def pair_step(p, acc, s=s):
        return acc + p_v[s * PAIR_PAD + p, :] * q_v[s * PAIR_PAD + p, :]
      acc = lax.fori_loop(0, NPAIR, pair_step, jnp.zeros((16,), jnp.float32),
                          unroll=5)
      acc_v[s, :] = acc
    pltpu.sync_copy(acc_v, ffm_out.at[pl.ds(base_s, S)])
    pltpu.sync_copy(e_v, emb_out.at[pl.ds(fe, EROWS)])
    pltpu.sync_copy(l_v, lin_out.at[pl.ds(fe, EROWS)])
    return carry

  lax.fori_loop(0, NCHUNK, chunk, 0)


def _sc_gather(wffm, wemb, wlin, idxp, idxq, xif):
  mesh = plsc.VectorSubcoreMesh(core_axis_name="c", subcore_axis_name="s")
  fn = pl.kernel(
      _sc_body,
      out_type=(
          jax.ShapeDtypeStruct((B, 16), jnp.float32),
          jax.ShapeDtypeStruct((B * F, D), jnp.float32),
          jax.ShapeDtypeStruct((B * F, 1), jnp.float32),
      ),
      mesh=mesh,
      scratch_types=(
          pltpu.VMEM((ROWS,), jnp.int32),
          pltpu.VMEM((ROWS,), jnp.int32),
          pltpu.VMEM((EROWS,), jnp.int32),
          pltpu.VMEM((ROWS, D), jnp.float32),
          pltpu.VMEM((ROWS, D), jnp.float32),
          pltpu.VMEM((EROWS, D), jnp.float32),
          pltpu.VMEM((EROWS, 1), jnp.float32),
          pltpu.VMEM((S, 16), jnp.float32),
          pltpu.SemaphoreType.DMA,
      ),
  )
  return fn(wffm, wemb, wlin, idxp, idxq, xif)


def _tc_body(emb, ffmacc, linv, w1, b1, g1, be1, w2, b2, g2, be2, w3, b3,
             blin, out):
  h = emb[...]
  h1 = jnp.dot(h, w1[...], preferred_element_type=jnp.float32,
               precision=lax.Precision.HIGHEST) + b1[...]
  mu1 = jnp.mean(h1, axis=0, keepdims=True)
  var1 = jnp.mean((h1 - mu1) ** 2, axis=0, keepdims=True)
  h1 = (h1 - mu1) / jnp.sqrt(var1 + 1e-5) * g1[...] + be1[...]
  h1 = jnp.maximum(h1, 0.0)
  h2 = jnp.dot(h1, w2[...], preferred_element_type=jnp.float32,
               precision=lax.Precision.HIGHEST) + b2[...]
  mu2 = jnp.mean(h2, axis=0, keepdims=True)
  var2 = jnp.mean((h2 - mu2) ** 2, axis=0, keepdims=True)
  h2 = (h2 - mu2) / jnp.sqrt(var2 + 1e-5) * g2[...] + be2[...]
  h2 = jnp.maximum(h2, 0.0)
  mlp = jnp.dot(h2, w3[...], preferred_element_type=jnp.float32,
                precision=lax.Precision.HIGHEST) + b3[...]
  lin = jnp.sum(linv[...], axis=1, keepdims=True) + blin[...]
  ffm = jnp.sum(ffmacc[...], axis=1, keepdims=True)
  out[...] = jax.nn.sigmoid(lin + ffm + mlp)


def _tc_head(emb, ffmacc, linv, w1, b1, g1, be1, w2, b2, g2, be2, w3, b3,
             blin):
  return pl.pallas_call(
      _tc_body,
      out_shape=jax.ShapeDtypeStruct((B, 1), jnp.float32),
  )(emb, ffmacc, linv, w1, b1, g1, be1, w2, b2, g2, be2, w3, b3, blin)


def kernel(x, offsets, W_embed, W_lin, b_lin, W_ffm, W1, b1, g1, be1, W2, b2,
           g2, be2, W3, b3):
  xi = x + offsets[None, :]
  wffm = W_ffm.reshape(F * V, D)
  colP = jnp.asarray(_colP)
  colQ = jnp.asarray(_colQ)
  baseP = jnp.asarray((_tabP.astype(np.int64) * V).astype(np.int32))
  baseQ = jnp.asarray((_tabQ.astype(np.int64) * V).astype(np.int32))
  idxp = (jnp.take(xi, colP, axis=1) + baseP[None, :]).reshape(-1)
  idxq = (jnp.take(xi, colQ, axis=1) + baseQ[None, :]).reshape(-1)
  xif = xi.reshape(-1)

  ffmacc, emb_rows, lin_rows = _sc_gather(wffm, W_embed, W_lin, idxp, idxq,
                                          xif)

  emb = emb_rows.reshape(B, EOD)
  linv = lin_rows.reshape(B, F)
  out = _tc_head(
      emb, ffmacc, linv,
      W1, b1.reshape(1, H1), g1.reshape(1, H1), be1.reshape(1, H1),
      W2, b2.reshape(1, H2), g2.reshape(1, H2), be2.reshape(1, H2),
      W3, b3.reshape(1, 1), b_lin.reshape(1, 1))
  return out.reshape(B)


# speedup vs baseline: 1.8360x; 1.8360x over previous
import functools

import jax
import jax.numpy as jnp
import numpy as np
from jax import lax
from jax.experimental import pallas as pl
from jax.experimental.pallas import tpu as pltpu
from jax.experimental.pallas import tpu_sc as plsc

B = 4096
F = 26
D = 16
FIELD = 3846
V = F * FIELD
H1, H2 = 256, 128
EOD = F * D

NPAIR = (F * (F - 1)) // 2
PAIR_PAD = 328

_iu, _ju = np.triu_indices(F, 1)
_colP = np.zeros(PAIR_PAD, np.int32)
_colP[:NPAIR] = _iu
_tabP = np.zeros(PAIR_PAD, np.int32)
_tabP[:NPAIR] = _ju
_colQ = np.zeros(PAIR_PAD, np.int32)
_colQ[:NPAIR] = _ju
_tabQ = np.zeros(PAIR_PAD, np.int32)
_tabQ[:NPAIR] = _iu

NW = 32
SPW = B // NW
S = 4
NCHUNK = SPW // S
ROWS = S * PAIR_PAD
NFULL = ROWS // 128
REM = ROWS - NFULL * 128
EROWS = S * F


def _sc_body(wffm, wemb, wlin, idxp, idxq, xif,
             ffm_out, emb_out,
             idxp_v, idxq_v, xif_v, p_v, q_v, e_v, l_v, acc_v, sem):
  cid = lax.axis_index("c")
  sid = lax.axis_index("s")
  wid = sid * 2 + cid

  def chunk(c, carry):
    base_s = wid * SPW + c * S
    fp = base_s * PAIR_PAD
    fe = base_s * F
    pltpu.sync_copy(idxp.at[pl.ds(fp, ROWS)], idxp_v)
    pltpu.sync_copy(idxq.at[pl.ds(fp, ROWS)], idxq_v)
    pltpu.sync_copy(xif.at[pl.ds(fe, EROWS)], xif_v)
    copies = []
    for k in range(NFULL):
      copies.append(pltpu.async_copy(
          wffm.at[idxp_v.at[pl.ds(k * 128, 128)]],
          p_v.at[pl.ds(k * 128, 128)], sem))
      copies.append(pltpu.async_copy(
          wffm.at[idxq_v.at[pl.ds(k * 128, 128)]],
          q_v.at[pl.ds(k * 128, 128)], sem))
    copies.append(pltpu.async_copy(
        wffm.at[idxp_v.at[pl.ds(NFULL * 128, REM)]],
        p_v.at[pl.ds(NFULL * 128, REM)], sem))
    copies.append(pltpu.async_copy(
        wffm.at[idxq_v.at[pl.ds(NFULL * 128, REM)]],
        q_v.at[pl.ds(NFULL * 128, REM)], sem))
    copies.append(pltpu.async_copy(wemb.at[xif_v], e_v, sem))
    copies.append(pltpu.async_copy(wlin.at[xif_v], l_v, sem))
    for cp in copies:
      cp.wait()
    for s in range(S):
      def pair_step(p, acc, s=s):
        return acc + p_v[s * PAIR_PAD + p, :] * q_v[s * PAIR_PAD + p, :]
      acc = lax.fori_loop(0, NPAIR, pair_step, jnp.zeros((16,), jnp.float32),
                          unroll=5)
      for i in range(F):
        acc = acc + l_v[s * F + i, :]
      acc_v[s, :] = acc
    pltpu.sync_copy(acc_v, ffm_out.at[pl.ds(base_s, S)])
    pltpu.sync_copy(e_v, emb_out.at[pl.ds(fe, EROWS)])
    return carry

  lax.fori_loop(0, NCHUNK, chunk, 0)


def _sc_gather(wffm, wemb, wlin, idxp, idxq, xif):
  mesh = plsc.VectorSubcoreMesh(core_axis_name="c", subcore_axis_name="s")
  fn = pl.kernel(
      _sc_body,
      out_type=(
          jax.ShapeDtypeStruct((B, 16), jnp.float32),
          jax.ShapeDtypeStruct((B * F, D), jnp.float32),
      ),
      mesh=mesh,
      compiler_params=pltpu.CompilerParams(use_tc_tiling_on_sc=False),
      scratch_types=(
          pltpu.VMEM((ROWS,), jnp.int32),
          pltpu.VMEM((ROWS,), jnp.int32),
          pltpu.VMEM((EROWS,), jnp.int32),
          pltpu.VMEM((ROWS, D), jnp.float32),
          pltpu.VMEM((ROWS, D), jnp.float32),
          pltpu.VMEM((EROWS, D), jnp.float32),
          pltpu.VMEM((EROWS, D), jnp.float32),
          pltpu.VMEM((S, 16), jnp.float32),
          pltpu.SemaphoreType.DMA,
      ),
  )
  return fn(wffm, wemb, wlin, idxp, idxq, xif)


def _tc_body(emb, ffmacc, w1, b1, g1, be1, w2, b2, g2, be2, w3, b3,
             blin, out):
  h = emb[...]
  h1 = jnp.dot(h, w1[...], preferred_element_type=jnp.float32,
               precision=lax.Precision.HIGHEST) + b1[...]
  mu1 = jnp.mean(h1, axis=0, keepdims=True)
  var1 = jnp.mean((h1 - mu1) ** 2, axis=0, keepdims=True)
  h1 = (h1 - mu1) / jnp.sqrt(var1 + 1e-5) * g1[...] + be1[...]
  h1 = jnp.maximum(h1, 0.0)
  h2 = jnp.dot(h1, w2[...], preferred_element_type=jnp.float32,
               precision=lax.Precision.HIGHEST) + b2[...]
  mu2 = jnp.mean(h2, axis=0, keepdims=True)
  var2 = jnp.mean((h2 - mu2) ** 2, axis=0, keepdims=True)
  h2 = (h2 - mu2) / jnp.sqrt(var2 + 1e-5) * g2[...] + be2[...]
  h2 = jnp.maximum(h2, 0.0)
  mlp = jnp.dot(h2, w3[...], preferred_element_type=jnp.float32,
                precision=lax.Precision.HIGHEST) + b3[...]
  linffm = jnp.sum(ffmacc[...], axis=1, keepdims=True) + blin[...]
  out[...] = jax.nn.sigmoid(linffm + mlp)


def _tc_head(emb, ffmacc, w1, b1, g1, be1, w2, b2, g2, be2, w3, b3,
             blin):
  return pl.pallas_call(
      _tc_body,
      out_shape=jax.ShapeDtypeStruct((B, 1), jnp.float32),
  )(emb, ffmacc, w1, b1, g1, be1, w2, b2, g2, be2, w3, b3, blin)


def kernel(x, offsets, W_embed, W_lin, b_lin, W_ffm, W1, b1, g1, be1, W2, b2,
           g2, be2, W3, b3):
  xi = x + offsets[None, :]
  wffm = W_ffm.reshape(F * V, D)
  colP = jnp.asarray(_colP)
  colQ = jnp.asarray(_colQ)
  baseP = jnp.asarray((_tabP.astype(np.int64) * V).astype(np.int32))
  baseQ = jnp.asarray((_tabQ.astype(np.int64) * V).astype(np.int32))
  idxp = (jnp.take(xi, colP, axis=1) + baseP[None, :]).reshape(-1)
  idxq = (jnp.take(xi, colQ, axis=1) + baseQ[None, :]).reshape(-1)
  xif = xi.reshape(-1)
  wlin16 = jnp.concatenate(
      [W_lin, jnp.zeros((V, D - 1), jnp.float32)], axis=1)

  ffmacc, emb_rows = _sc_gather(wffm, W_embed, wlin16, idxp, idxq, xif)

  emb = emb_rows.reshape(B, EOD)
  out = _tc_head(
      emb, ffmacc,
      W1, b1.reshape(1, H1), g1.reshape(1, H1), be1.reshape(1, H1),
      W2, b2.reshape(1, H2), g2.reshape(1, H2), be2.reshape(1, H2),
      W3, b3.reshape(1, 1), b_lin.reshape(1, 1))
  return out.reshape(B)


# speedup vs baseline: 1.8510x; 1.0082x over previous
import functools

import jax
import jax.numpy as jnp
import numpy as np
from jax import lax
from jax.experimental import pallas as pl
from jax.experimental.pallas import tpu as pltpu
from jax.experimental.pallas import tpu_sc as plsc

B = 4096
F = 26
D = 16
FIELD = 3846
V = F * FIELD
H1, H2 = 256, 128
EOD = F * D

NPAIR = (F * (F - 1)) // 2
PAIR_PAD = 336
NGRP = PAIR_PAD // 16

_iu, _ju = np.triu_indices(F, 1)
_colP = np.zeros(PAIR_PAD, np.int32)
_colP[:NPAIR] = _iu
_tabP = np.zeros(PAIR_PAD, np.int32)
_tabP[:NPAIR] = _ju
_colQ = np.zeros(PAIR_PAD, np.int32)
_colQ[:NPAIR] = _ju
_tabQ = np.zeros(PAIR_PAD, np.int32)
_tabQ[:NPAIR] = _iu

NW = 32
SPW = B // NW
S = 4
NCHUNK = SPW // S
ROWS = S * PAIR_PAD
NFULL = ROWS // 128
REM = ROWS - NFULL * 128
EROWS = S * F

_colP4 = np.concatenate([_colP + s * F for s in range(S)]).astype(np.int32)
_colQ4 = np.concatenate([_colQ + s * F for s in range(S)]).astype(np.int32)
_baseP4 = np.tile((_tabP.astype(np.int64) * V).astype(np.int32), S)
_baseQ4 = np.tile((_tabQ.astype(np.int64) * V).astype(np.int32), S)


def _sc_body(wffm, wemb, wlin, colp, colq, basep, baseq, xif,
             ffm_out, emb_out,
             idxp_v, idxq_v, xif_v, colp_v, colq_v, basep_v, baseq_v,
             p_v, q_v, e_v, l_v, acc_v, sem):
  cid = lax.axis_index("c")
  sid = lax.axis_index("s")
  wid = sid * 2 + cid

  pltpu.sync_copy(colp, colp_v)
  pltpu.sync_copy(colq, colq_v)
  pltpu.sync_copy(basep, basep_v)
  pltpu.sync_copy(baseq, baseq_v)

  def chunk(c, carry):
    base_s = wid * SPW + c * S
    fe = base_s * F
    pltpu.sync_copy(xif.at[pl.ds(fe, EROWS)], xif_v)
    for g in range(S * NGRP):
      sl = pl.ds(g * 16, 16)
      idxp_v[sl] = plsc.load_gather(xif_v, [colp_v[sl]]) + basep_v[sl]
      idxq_v[sl] = plsc.load_gather(xif_v, [colq_v[sl]]) + baseq_v[sl]
    copies = []
    for k in range(NFULL):
      copies.append(pltpu.async_copy(
          wffm.at[idxp_v.at[pl.ds(k * 128, 128)]],
          p_v.at[pl.ds(k * 128, 128)], sem))
      copies.append(pltpu.async_copy(
          wffm.at[idxq_v.at[pl.ds(k * 128, 128)]],
          q_v.at[pl.ds(k * 128, 128)], sem))
    copies.append(pltpu.async_copy(
        wffm.at[idxp_v.at[pl.ds(NFULL * 128, REM)]],
        p_v.at[pl.ds(NFULL * 128, REM)], sem))
    copies.append(pltpu.async_copy(
        wffm.at[idxq_v.at[pl.ds(NFULL * 128, REM)]],
        q_v.at[pl.ds(NFULL * 128, REM)], sem))
    copies.append(pltpu.async_copy(wemb.at[xif_v], e_v, sem))
    copies.append(pltpu.async_copy(wlin.at[xif_v], l_v, sem))
    for cp in copies:
      cp.wait()
    for s in range(S):
      def pair_step(p, acc, s=s):
        return acc + p_v[s * PAIR_PAD + p, :] * q_v[s * PAIR_PAD + p, :]
      acc = lax.fori_loop(0, NPAIR, pair_step, jnp.zeros((16,), jnp.float32),
                          unroll=5)
      for i in range(F):
        acc = acc + l_v[s * F + i, :]
      acc_v[s, :] = acc
    pltpu.sync_copy(acc_v, ffm_out.at[pl.ds(base_s, S)])
    pltpu.sync_copy(e_v, emb_out.at[pl.ds(fe, EROWS)])
    return carry

  lax.fori_loop(0, NCHUNK, chunk, 0)


def _sc_gather(wffm, wemb, wlin, colp, colq, basep, baseq, xif):
  mesh = plsc.VectorSubcoreMesh(core_axis_name="c", subcore_axis_name="s")
  fn = pl.kernel(
      _sc_body,
      out_type=(
          jax.ShapeDtypeStruct((B, 16), jnp.float32),
          jax.ShapeDtypeStruct((B * F, D), jnp.float32),
      ),
      mesh=mesh,
      compiler_params=pltpu.CompilerParams(
          use_tc_tiling_on_sc=False, needs_layout_passes=False),
      scratch_types=(
          pltpu.VMEM((ROWS,), jnp.int32),
          pltpu.VMEM((ROWS,), jnp.int32),
          pltpu.VMEM((EROWS,), jnp.int32),
          pltpu.VMEM((ROWS,), jnp.int32),
          pltpu.VMEM((ROWS,), jnp.int32),
          pltpu.VMEM((ROWS,), jnp.int32),
          pltpu.VMEM((ROWS,), jnp.int32),
          pltpu.VMEM((ROWS, D), jnp.float32),
          pltpu.VMEM((ROWS, D), jnp.float32),
          pltpu.VMEM((EROWS, D), jnp.float32),
          pltpu.VMEM((EROWS, D), jnp.float32),
          pltpu.VMEM((S, 16), jnp.float32),
          pltpu.SemaphoreType.DMA,
      ),
  )
  return fn(wffm, wemb, wlin, colp, colq, basep, baseq, xif)


def _tc_body(emb, ffmacc, w1, b1, g1, be1, w2, b2, g2, be2, w3, b3,
             blin, out):
  h = emb[...]
  h1 = jnp.dot(h, w1[...], preferred_element_type=jnp.float32,
               precision=lax.Precision.HIGHEST) + b1[...]
  mu1 = jnp.mean(h1, axis=0, keepdims=True)
  var1 = jnp.mean((h1 - mu1) ** 2, axis=0, keepdims=True)
  h1 = (h1 - mu1) / jnp.sqrt(var1 + 1e-5) * g1[...] + be1[...]
  h1 = jnp.maximum(h1, 0.0)
  h2 = jnp.dot(h1, w2[...], preferred_element_type=jnp.float32,
               precision=lax.Precision.HIGHEST) + b2[...]
  mu2 = jnp.mean(h2, axis=0, keepdims=True)
  var2 = jnp.mean((h2 - mu2) ** 2, axis=0, keepdims=True)
  h2 = (h2 - mu2) / jnp.sqrt(var2 + 1e-5) * g2[...] + be2[...]
  h2 = jnp.maximum(h2, 0.0)
  mlp = jnp.dot(h2, w3[...], preferred_element_type=jnp.float32,
                precision=lax.Precision.HIGHEST) + b3[...]
  linffm = jnp.sum(ffmacc[...], axis=1, keepdims=True) + blin[...]
  out[...] = jax.nn.sigmoid(linffm + mlp)


def _tc_head(emb, ffmacc, w1, b1, g1, be1, w2, b2, g2, be2, w3, b3,
             blin):
  return pl.pallas_call(
      _tc_body,
      out_shape=jax.ShapeDtypeStruct((B, 1), jnp.float32),
  )(emb, ffmacc, w1, b1, g1, be1, w2, b2, g2, be2, w3, b3, blin)


def kernel(x, offsets, W_embed, W_lin, b_lin, W_ffm, W1, b1, g1, be1, W2, b2,
           g2, be2, W3, b3):
  xi = x + offsets[None, :]
  wffm = W_ffm.reshape(F * V, D)
  xif = xi.reshape(-1)
  wlin16 = jnp.concatenate(
      [W_lin, jnp.zeros((V, D - 1), jnp.float32)], axis=1)

  ffmacc, emb_rows = _sc_gather(
      wffm, W_embed, wlin16,
      jnp.asarray(_colP4), jnp.asarray(_colQ4),
      jnp.asarray(_baseP4), jnp.asarray(_baseQ4), xif)

  emb = emb_rows.reshape(B, EOD)
  out = _tc_head(
      emb, ffmacc,
      W1, b1.reshape(1, H1), g1.reshape(1, H1), be1.reshape(1, H1),
      W2, b2.reshape(1, H2), g2.reshape(1, H2), be2.reshape(1, H2),
      W3, b3.reshape(1, 1), b_lin.reshape(1, 1))
  return out.reshape(B)


# speedup vs baseline: 10.5218x; 5.6845x over previous
import jax
import jax.numpy as jnp
import numpy as np
from jax import lax
from jax.experimental import pallas as pl
from jax.experimental.pallas import tpu as pltpu
from jax.experimental.pallas import tpu_sc as plsc

B = 4096
F = 26
D = 16
FIELD = 3846
V = F * FIELD
H1, H2 = 256, 128
EOD = F * D
WIDE = 512
ECOL = EOD
LCOL = EOD + D

_PAIRS = [(i, j) for i in range(F - 1) for j in range(i + 1, F)]

NW = 32
SPW = B // NW
S = 4
NCHUNK = SPW // S
GROWS = S * F


def _sc_body(big, xif, ffm_out, emb_out, xifw_v, g_v, e_v, acc_v, sem):
  cid = lax.axis_index("c")
  sid = lax.axis_index("s")
  wid = sid * 2 + cid

  pltpu.sync_copy(xif.at[pl.ds(wid * SPW * F, SPW * F)], xifw_v)

  def chunk(c, carry):
    base_s = wid * SPW + c * S
    pltpu.async_copy(
        big.at[xifw_v.at[pl.ds(c * GROWS, GROWS)]], g_v, sem).wait()
    def sample(s, carry2):
      r0 = s * F
      acc = jnp.zeros((16,), jnp.float32)
      for (i, j) in _PAIRS:
        acc = acc + (g_v[r0 + i, pl.ds(16 * j, 16)] *
                     g_v[r0 + j, pl.ds(16 * i, 16)])
      for f in range(F):
        acc = acc + g_v[r0 + f, pl.ds(LCOL, 16)]
        e_v[s, pl.ds(f * D, D)] = g_v[r0 + f, pl.ds(ECOL, D)]
      acc_v[s, :] = acc
      return carry2

    lax.fori_loop(0, S, sample, 0)
    pltpu.sync_copy(acc_v, ffm_out.at[pl.ds(base_s, S)])
    pltpu.sync_copy(e_v, emb_out.at[pl.ds(base_s, S)])
    return carry

  lax.fori_loop(0, NCHUNK, chunk, 0)


def _sc_gather(big, xif):
  mesh = plsc.VectorSubcoreMesh(core_axis_name="c", subcore_axis_name="s")
  fn = pl.kernel(
      _sc_body,
      out_type=(
          jax.ShapeDtypeStruct((B, 16), jnp.float32),
          jax.ShapeDtypeStruct((B, EOD), jnp.float32),
      ),
      mesh=mesh,
      compiler_params=pltpu.CompilerParams(use_tc_tiling_on_sc=True),
      scratch_types=(
          pltpu.VMEM((SPW * F,), jnp.int32),
          pltpu.VMEM((GROWS, WIDE), jnp.float32),
          pltpu.VMEM((S, EOD), jnp.float32),
          pltpu.VMEM((S, 16), jnp.float32),
          pltpu.SemaphoreType.DMA,
      ),
  )
  return fn(big, xif)


def _tc_body(emb, ffmacc, w1, b1, g1, be1, w2, b2, g2, be2, w3, b3,
             blin, out):
  h = emb[...]
  h1 = jnp.dot(h, w1[...], preferred_element_type=jnp.float32,
               precision=lax.Precision.HIGHEST) + b1[...]
  mu1 = jnp.mean(h1, axis=0, keepdims=True)
  var1 = jnp.mean((h1 - mu1) ** 2, axis=0, keepdims=True)
  h1 = (h1 - mu1) / jnp.sqrt(var1 + 1e-5) * g1[...] + be1[...]
  h1 = jnp.maximum(h1, 0.0)
  h2 = jnp.dot(h1, w2[...], preferred_element_type=jnp.float32,
               precision=lax.Precision.HIGHEST) + b2[...]
  mu2 = jnp.mean(h2, axis=0, keepdims=True)
  var2 = jnp.mean((h2 - mu2) ** 2, axis=0, keepdims=True)
  h2 = (h2 - mu2) / jnp.sqrt(var2 + 1e-5) * g2[...] + be2[...]
  h2 = jnp.maximum(h2, 0.0)
  mlp = jnp.dot(h2, w3[...], preferred_element_type=jnp.float32,
                precision=lax.Precision.HIGHEST) + b3[...]
  linffm = jnp.sum(ffmacc[...], axis=1, keepdims=True) + blin[...]
  out[...] = jax.nn.sigmoid(linffm + mlp)


def _tc_head(emb, ffmacc, w1, b1, g1, be1, w2, b2, g2, be2, w3, b3, blin):
  return pl.pallas_call(
      _tc_body,
      out_shape=jax.ShapeDtypeStruct((B, 1), jnp.float32),
  )(emb, ffmacc, w1, b1, g1, be1, w2, b2, g2, be2, w3, b3, blin)


def kernel(x, offsets, W_embed, W_lin, b_lin, W_ffm, W1, b1, g1, be1, W2, b2,
           g2, be2, W3, b3):
  xi = x + offsets[None, :]
  xif = xi.reshape(-1)
  big = jnp.concatenate(
      [jnp.transpose(W_ffm, (1, 0, 2)).reshape(V, EOD),
       W_embed, W_lin,
       jnp.zeros((V, WIDE - EOD - D - 1), jnp.float32)], axis=1)

  ffmacc, emb = _sc_gather(big, xif)

  out = _tc_head(
      emb, ffmacc,
      W1, b1.reshape(1, H1), g1.reshape(1, H1), be1.reshape(1, H1),
      W2, b2.reshape(1, H2), g2.reshape(1, H2), be2.reshape(1, H2),
      W3, b3.reshape(1, 1), b_lin.reshape(1, 1))
  return out.reshape(B)


# speedup vs baseline: 27.3139x; 2.5959x over previous
import jax
import jax.numpy as jnp
import numpy as np
from jax import lax
from jax.experimental import pallas as pl
from jax.experimental.pallas import tpu as pltpu
from jax.experimental.pallas import tpu_sc as plsc

B = 4096
F = 26
D = 16
FIELD = 3846
V = F * FIELD
H1, H2 = 256, 128
EOD = F * D
WIDE = 512
ECOL = EOD
LCOL = EOD + D

_PAIRS = [(i, j) for i in range(F - 1) for j in range(i + 1, F)]

_SEL = np.zeros((EOD + D + 1, WIDE), np.float32)
for _c in range(EOD + D + 1):
  _SEL[_c, _c] = 1.0

NW = 32
SPW = B // NW
S = 4
NCHUNK = SPW // S
GROWS = S * F


def _sc_body(big, xif, ffm_out, emb_out, xifw_v, g_v, e_v, acc_v, sem):
  cid = lax.axis_index("c")
  sid = lax.axis_index("s")
  wid = sid * 2 + cid

  pltpu.sync_copy(xif.at[pl.ds(wid * SPW * F, SPW * F)], xifw_v)

  def chunk(c, carry):
    base_s = wid * SPW + c * S
    pltpu.async_copy(
        big.at[xifw_v.at[pl.ds(c * GROWS, GROWS)]], g_v, sem).wait()
    def sample(s, carry2):
      r0 = s * F
      acc = jnp.zeros((16,), jnp.float32)
      for (i, j) in _PAIRS:
        acc = acc + (g_v[r0 + i, pl.ds(16 * j, 16)] *
                     g_v[r0 + j, pl.ds(16 * i, 16)])
      for f in range(F):
        acc = acc + g_v[r0 + f, pl.ds(LCOL, 16)]
        e_v[s, pl.ds(f * D, D)] = g_v[r0 + f, pl.ds(ECOL, D)]
      acc_v[s, :] = acc
      return carry2

    lax.fori_loop(0, S, sample, 0)
    pltpu.sync_copy(acc_v, ffm_out.at[pl.ds(base_s, S)])
    pltpu.sync_copy(e_v, emb_out.at[pl.ds(base_s, S)])
    return carry

  lax.fori_loop(0, NCHUNK, chunk, 0)


def _sc_gather(big, xif):
  mesh = plsc.VectorSubcoreMesh(core_axis_name="c", subcore_axis_name="s")
  fn = pl.kernel(
      _sc_body,
      out_type=(
          jax.ShapeDtypeStruct((B, 16), jnp.float32),
          jax.ShapeDtypeStruct((B, EOD), jnp.float32),
      ),
      mesh=mesh,
      compiler_params=pltpu.CompilerParams(use_tc_tiling_on_sc=True),
      scratch_types=(
          pltpu.VMEM((SPW * F,), jnp.int32),
          pltpu.VMEM((GROWS, WIDE), jnp.float32),
          pltpu.VMEM((S, EOD), jnp.float32),
          pltpu.VMEM((S, 16), jnp.float32),
          pltpu.SemaphoreType.DMA,
      ),
  )
  return fn(big, xif)


def _tc_body(emb, ffmacc, w1, b1, g1, be1, w2, b2, g2, be2, w3, b3,
             blin, out):
  h = emb[...]
  h1 = jnp.dot(h, w1[...], preferred_element_type=jnp.float32,
               precision=lax.Precision.HIGHEST) + b1[...]
  mu1 = jnp.mean(h1, axis=0, keepdims=True)
  var1 = jnp.mean((h1 - mu1) ** 2, axis=0, keepdims=True)
  h1 = (h1 - mu1) / jnp.sqrt(var1 + 1e-5) * g1[...] + be1[...]
  h1 = jnp.maximum(h1, 0.0)
  h2 = jnp.dot(h1, w2[...], preferred_element_type=jnp.float32,
               precision=lax.Precision.HIGHEST) + b2[...]
  mu2 = jnp.mean(h2, axis=0, keepdims=True)
  var2 = jnp.mean((h2 - mu2) ** 2, axis=0, keepdims=True)
  h2 = (h2 - mu2) / jnp.sqrt(var2 + 1e-5) * g2[...] + be2[...]
  h2 = jnp.maximum(h2, 0.0)
  mlp = jnp.dot(h2, w3[...], preferred_element_type=jnp.float32,
                precision=lax.Precision.HIGHEST) + b3[...]
  linffm = jnp.sum(ffmacc[...], axis=1, keepdims=True) + blin[...]
  out[...] = jax.nn.sigmoid(linffm + mlp)


def _tc_head(emb, ffmacc, w1, b1, g1, be1, w2, b2, g2, be2, w3, b3, blin):
  return pl.pallas_call(
      _tc_body,
      out_shape=jax.ShapeDtypeStruct((B, 1), jnp.float32),
  )(emb, ffmacc, w1, b1, g1, be1, w2, b2, g2, be2, w3, b3, blin)


def kernel(x, offsets, W_embed, W_lin, b_lin, W_ffm, W1, b1, g1, be1, W2, b2,
           g2, be2, W3, b3):
  xi = x + offsets[None, :]
  xif = xi.reshape(-1)
  stack = jnp.concatenate(
      [jnp.transpose(W_ffm, (0, 2, 1)).reshape(EOD, V),
       W_embed.T, W_lin.T], axis=0)
  big = lax.dot_general(stack, jnp.asarray(_SEL),
                        (((0,), (0,)), ((), ())))

  ffmacc, emb = _sc_gather(big, xif)

  out = _tc_head(
      emb, ffmacc,
      W1, b1.reshape(1, H1), g1.reshape(1, H1), be1.reshape(1, H1),
      W2, b2.reshape(1, H2), g2.reshape(1, H2), be2.reshape(1, H2),
      W3, b3.reshape(1, 1), b_lin.reshape(1, 1))
  return out.reshape(B)


# speedup vs baseline: 34.2760x; 1.2549x over previous
import jax
import jax.numpy as jnp
import numpy as np
from jax import lax
from jax.experimental import pallas as pl
from jax.experimental.pallas import tpu as pltpu
from jax.experimental.pallas import tpu_sc as plsc

B = 4096
F = 26
D = 16
FIELD = 3846
V = F * FIELD
H1, H2 = 256, 128
EOD = F * D
WIDE = 512
ECOL = EOD
LCOL = EOD + D

_PAIRS = [(i, j) for i in range(F - 1) for j in range(i + 1, F)]

_SELF = np.zeros((EOD, WIDE), np.float32)
for _c in range(EOD):
  _SELF[_c, _c] = 1.0
_SELE = np.zeros((D, WIDE), np.float32)
for _c in range(D):
  _SELE[_c, ECOL + _c] = 1.0
_SELL = np.zeros((1, WIDE), np.float32)
_SELL[0, LCOL] = 1.0

VB = 2048
NVB = -(-V // VB)

NW = 32
SPW = B // NW
S = 4
NCHUNK = SPW // S
GROWS = S * F


def _sc_body(big, xif, ffm_out, emb_out, xifw_v, g_v, e_v, acc_v, sem):
  cid = lax.axis_index("c")
  sid = lax.axis_index("s")
  wid = sid * 2 + cid

  pltpu.sync_copy(xif.at[pl.ds(wid * SPW * F, SPW * F)], xifw_v)

  def chunk(c, carry):
    base_s = wid * SPW + c * S
    pltpu.async_copy(
        big.at[xifw_v.at[pl.ds(c * GROWS, GROWS)]], g_v, sem).wait()
    def sample(s, carry2):
      r0 = s * F
      acc = jnp.zeros((16,), jnp.float32)
      for (i, j) in _PAIRS:
        acc = acc + (g_v[r0 + i, pl.ds(16 * j, 16)] *
                     g_v[r0 + j, pl.ds(16 * i, 16)])
      for f in range(F):
        acc = acc + g_v[r0 + f, pl.ds(LCOL, 16)]
        e_v[s, pl.ds(f * D, D)] = g_v[r0 + f, pl.ds(ECOL, D)]
      acc_v[s, :] = acc
      return carry2

    lax.fori_loop(0, S, sample, 0)
    pltpu.sync_copy(acc_v, ffm_out.at[pl.ds(base_s, S)])
    pltpu.sync_copy(e_v, emb_out.at[pl.ds(base_s, S)])
    return carry

  lax.fori_loop(0, NCHUNK, chunk, 0)


def _sc_gather(big, xif):
  mesh = plsc.VectorSubcoreMesh(core_axis_name="c", subcore_axis_name="s")
  fn = pl.kernel(
      _sc_body,
      out_type=(
          jax.ShapeDtypeStruct((B, 16), jnp.float32),
          jax.ShapeDtypeStruct((B, EOD), jnp.float32),
      ),
      mesh=mesh,
      compiler_params=pltpu.CompilerParams(use_tc_tiling_on_sc=True),
      scratch_types=(
          pltpu.VMEM((SPW * F,), jnp.int32),
          pltpu.VMEM((GROWS, WIDE), jnp.float32),
          pltpu.VMEM((S, EOD), jnp.float32),
          pltpu.VMEM((S, 16), jnp.float32),
          pltpu.SemaphoreType.DMA,
      ),
  )
  return fn(big, xif)


def _pack_body(a3, wembt, wlint, self_, sele, sell, big):
  dn = (((0,), (0,)), ((), ()))
  acc = lax.dot_general(a3[...], self_[...], dn,
                        preferred_element_type=jnp.float32)
  acc = acc + lax.dot_general(wembt[...], sele[...], dn,
                              preferred_element_type=jnp.float32)
  acc = acc + lax.dot_general(wlint[...], sell[...], dn,
                              preferred_element_type=jnp.float32)
  big[...] = acc


def _pack(a3, wembt, wlint):
  return pl.pallas_call(
      _pack_body,
      grid=(NVB,),
      in_specs=[
          pl.BlockSpec((EOD, VB), lambda v: (0, v)),
          pl.BlockSpec((D, VB), lambda v: (0, v)),
          pl.BlockSpec((1, VB), lambda v: (0, v)),
          pl.BlockSpec((EOD, WIDE), lambda v: (0, 0)),
          pl.BlockSpec((D, WIDE), lambda v: (0, 0)),
          pl.BlockSpec((1, WIDE), lambda v: (0, 0)),
      ],
      out_specs=pl.BlockSpec((VB, WIDE), lambda v: (v, 0)),
      out_shape=jax.ShapeDtypeStruct((NVB * VB, WIDE), jnp.float32),
  )(a3, wembt, wlint, jnp.asarray(_SELF), jnp.asarray(_SELE),
    jnp.asarray(_SELL))


def _tc_body(emb, ffmacc, w1, b1, g1, be1, w2, b2, g2, be2, w3, b3,
             blin, out):
  h = emb[...]
  h1 = jnp.dot(h, w1[...], preferred_element_type=jnp.float32,
               precision=lax.Precision.HIGHEST) + b1[...]
  mu1 = jnp.mean(h1, axis=0, keepdims=True)
  var1 = jnp.mean((h1 - mu1) ** 2, axis=0, keepdims=True)
  h1 = (h1 - mu1) / jnp.sqrt(var1 + 1e-5) * g1[...] + be1[...]
  h1 = jnp.maximum(h1, 0.0)
  h2 = jnp.dot(h1, w2[...], preferred_element_type=jnp.float32,
               precision=lax.Precision.HIGHEST) + b2[...]
  mu2 = jnp.mean(h2, axis=0, keepdims=True)
  var2 = jnp.mean((h2 - mu2) ** 2, axis=0, keepdims=True)
  h2 = (h2 - mu2) / jnp.sqrt(var2 + 1e-5) * g2[...] + be2[...]
  h2 = jnp.maximum(h2, 0.0)
  mlp = jnp.dot(h2, w3[...], preferred_element_type=jnp.float32,
                precision=lax.Precision.HIGHEST) + b3[...]
  linffm = jnp.sum(ffmacc[...], axis=1, keepdims=True) + blin[...]
  out[...] = jax.nn.sigmoid(linffm + mlp)


def _tc_head(emb, ffmacc, w1, b1, g1, be1, w2, b2, g2, be2, w3, b3, blin):
  return pl.pallas_call(
      _tc_body,
      out_shape=jax.ShapeDtypeStruct((B, 1), jnp.float32),
  )(emb, ffmacc, w1, b1, g1, be1, w2, b2, g2, be2, w3, b3, blin)


def kernel(x, offsets, W_embed, W_lin, b_lin, W_ffm, W1, b1, g1, be1, W2, b2,
           g2, be2, W3, b3):
  xi = x + offsets[None, :]
  xif = xi.reshape(-1)
  big = _pack(jnp.transpose(W_ffm, (0, 2, 1)).reshape(EOD, V),
              W_embed.T, W_lin.T)

  ffmacc, emb = _sc_gather(big, xif)

  out = _tc_head(
      emb, ffmacc,
      W1, b1.reshape(1, H1), g1.reshape(1, H1), be1.reshape(1, H1),
      W2, b2.reshape(1, H2), g2.reshape(1, H2), be2.reshape(1, H2),
      W3, b3.reshape(1, 1), b_lin.reshape(1, 1))
  return out.reshape(B)


# speedup vs baseline: 40.6217x; 1.1851x over previous
import jax
import jax.numpy as jnp
import numpy as np
from jax import lax
from jax.experimental import pallas as pl
from jax.experimental.pallas import tpu as pltpu
from jax.experimental.pallas import tpu_sc as plsc

B = 4096
F = 26
D = 16
FIELD = 3846
V = F * FIELD
H1, H2 = 256, 128
EOD = F * D
WIDE = 512
ECOL = EOD
LCOL = EOD + D

_PAIRS = [(i, j) for i in range(F - 1) for j in range(i + 1, F)]

_SELF = np.zeros((EOD, WIDE), np.float32)
for _c in range(EOD):
  _SELF[_c, _c] = 1.0
_SELE = np.zeros((D, WIDE), np.float32)
for _c in range(D):
  _SELE[_c, ECOL + _c] = 1.0
_SELL = np.zeros((1, WIDE), np.float32)
_SELL[0, LCOL] = 1.0

VB = 2048
NVB = -(-V // VB)

NW = 32
SPW = B // NW
S = 4
NCHUNK = SPW // S
GROWS = S * F


def _sc_body(big, xif, ffm_out, emb_out, xifw_v, g0_v, g1_v, e_v, acc_v,
             sem0, sem1):
  cid = lax.axis_index("c")
  sid = lax.axis_index("s")
  wid = sid * 2 + cid

  pltpu.sync_copy(xif.at[pl.ds(wid * SPW * F, SPW * F)], xifw_v)

  def start(c, g_v, sem):
    pltpu.make_async_copy(
        big.at[xifw_v.at[pl.ds(c * GROWS, GROWS)]], g_v, sem).start()

  def finish_and_compute(c, g_v, sem):
    base_s = wid * SPW + c * S
    pltpu.make_async_copy(
        big.at[xifw_v.at[pl.ds(c * GROWS, GROWS)]], g_v, sem).wait()

    def sample(s, carry2):
      r0 = s * F
      acc = jnp.zeros((16,), jnp.float32)
      for (i, j) in _PAIRS:
        acc = acc + (g_v[r0 + i, pl.ds(16 * j, 16)] *
                     g_v[r0 + j, pl.ds(16 * i, 16)])
      for f in range(F):
        acc = acc + g_v[r0 + f, pl.ds(LCOL, 16)]
        e_v[s, pl.ds(f * D, D)] = g_v[r0 + f, pl.ds(ECOL, D)]
      acc_v[s, :] = acc
      return carry2

    lax.fori_loop(0, S, sample, 0)
    pltpu.sync_copy(acc_v, ffm_out.at[pl.ds(base_s, S)])
    pltpu.sync_copy(e_v, emb_out.at[pl.ds(base_s, S)])

  start(0, g0_v, sem0)

  def pair_of_chunks(nn, carry):
    c0 = 2 * nn
    start(c0 + 1, g1_v, sem1)
    finish_and_compute(c0, g0_v, sem0)

    @pl.when(c0 + 2 < NCHUNK)
    def _():
      start(c0 + 2, g0_v, sem0)

    finish_and_compute(c0 + 1, g1_v, sem1)
    return carry

  lax.fori_loop(0, NCHUNK // 2, pair_of_chunks, 0)


def _sc_gather(big, xif):
  mesh = plsc.VectorSubcoreMesh(core_axis_name="c", subcore_axis_name="s")
  fn = pl.kernel(
      _sc_body,
      out_type=(
          jax.ShapeDtypeStruct((B, 16), jnp.float32),
          jax.ShapeDtypeStruct((B, EOD), jnp.float32),
      ),
      mesh=mesh,
      compiler_params=pltpu.CompilerParams(use_tc_tiling_on_sc=True),
      scratch_types=(
          pltpu.VMEM((SPW * F,), jnp.int32),
          pltpu.VMEM((GROWS, WIDE), jnp.float32),
          pltpu.VMEM((GROWS, WIDE), jnp.float32),
          pltpu.VMEM((S, EOD), jnp.float32),
          pltpu.VMEM((S, 16), jnp.float32),
          pltpu.SemaphoreType.DMA,
          pltpu.SemaphoreType.DMA,
      ),
  )
  return fn(big, xif)


def _pack_body(a3, wembt, wlint, self_, sele, sell, big):
  dn = (((0,), (0,)), ((), ()))
  acc = lax.dot_general(a3[...], self_[...], dn,
                        preferred_element_type=jnp.float32)
  acc = acc + lax.dot_general(wembt[...], sele[...], dn,
                              preferred_element_type=jnp.float32)
  acc = acc + lax.dot_general(wlint[...], sell[...], dn,
                              preferred_element_type=jnp.float32)
  big[...] = acc


def _pack(a3, wembt, wlint):
  return pl.pallas_call(
      _pack_body,
      grid=(NVB,),
      in_specs=[
          pl.BlockSpec((EOD, VB), lambda v: (0, v)),
          pl.BlockSpec((D, VB), lambda v: (0, v)),
          pl.BlockSpec((1, VB), lambda v: (0, v)),
          pl.BlockSpec((EOD, WIDE), lambda v: (0, 0)),
          pl.BlockSpec((D, WIDE), lambda v: (0, 0)),
          pl.BlockSpec((1, WIDE), lambda v: (0, 0)),
      ],
      out_specs=pl.BlockSpec((VB, WIDE), lambda v: (v, 0)),
      out_shape=jax.ShapeDtypeStruct((NVB * VB, WIDE), jnp.float32),
  )(a3, wembt, wlint, jnp.asarray(_SELF), jnp.asarray(_SELE),
    jnp.asarray(_SELL))


def _tc_body(emb, ffmacc, w1, b1, g1, be1, w2, b2, g2, be2, w3, b3,
             blin, out):
  h = emb[...]
  h1 = jnp.dot(h, w1[...], preferred_element_type=jnp.float32,
               precision=lax.Precision.HIGHEST) + b1[...]
  mu1 = jnp.mean(h1, axis=0, keepdims=True)
  var1 = jnp.mean((h1 - mu1) ** 2, axis=0, keepdims=True)
  h1 = (h1 - mu1) / jnp.sqrt(var1 + 1e-5) * g1[...] + be1[...]
  h1 = jnp.maximum(h1, 0.0)
  h2 = jnp.dot(h1, w2[...], preferred_element_type=jnp.float32,
               precision=lax.Precision.HIGHEST) + b2[...]
  mu2 = jnp.mean(h2, axis=0, keepdims=True)
  var2 = jnp.mean((h2 - mu2) ** 2, axis=0, keepdims=True)
  h2 = (h2 - mu2) / jnp.sqrt(var2 + 1e-5) * g2[...] + be2[...]
  h2 = jnp.maximum(h2, 0.0)
  mlp = jnp.dot(h2, w3[...], preferred_element_type=jnp.float32,
                precision=lax.Precision.HIGHEST) + b3[...]
  linffm = jnp.sum(ffmacc[...], axis=1, keepdims=True) + blin[...]
  out[...] = jax.nn.sigmoid(linffm + mlp)


def _tc_head(emb, ffmacc, w1, b1, g1, be1, w2, b2, g2, be2, w3, b3, blin):
  return pl.pallas_call(
      _tc_body,
      out_shape=jax.ShapeDtypeStruct((B, 1), jnp.float32),
  )(emb, ffmacc, w1, b1, g1, be1, w2, b2, g2, be2, w3, b3, blin)


def kernel(x, offsets, W_embed, W_lin, b_lin, W_ffm, W1, b1, g1, be1, W2, b2,
           g2, be2, W3, b3):
  xi = x + offsets[None, :]
  xif = xi.reshape(-1)
  big = _pack(jnp.transpose(W_ffm, (0, 2, 1)).reshape(EOD, V),
              W_embed.T, W_lin.T)

  ffmacc, emb = _sc_gather(big, xif)

  out = _tc_head(
      emb, ffmacc,
      W1, b1.reshape(1, H1), g1.reshape(1, H1), be1.reshape(1, H1),
      W2, b2.reshape(1, H2), g2.reshape(1, H2), be2.reshape(1, H2),
      W3, b3.reshape(1, 1), b_lin.reshape(1, 1))
  return out.reshape(B)


# speedup vs baseline: 45.1420x; 1.1113x over previous
import jax
import jax.numpy as jnp
import numpy as np
from jax import lax
from jax.experimental import pallas as pl
from jax.experimental.pallas import tpu as pltpu
from jax.experimental.pallas import tpu_sc as plsc

B = 4096
F = 26
D = 16
FIELD = 3846
V = F * FIELD
H1, H2 = 256, 128
EOD = F * D
WIDE = 512
ECOL = EOD
LCOL = EOD + D

_PAIRS = [(i, j) for i in range(F - 1) for j in range(i + 1, F)]

_SELF = np.zeros((EOD, WIDE), np.float32)
for _c in range(EOD):
  _SELF[_c, _c] = 1.0
_SELE = np.zeros((D, WIDE), np.float32)
for _c in range(D):
  _SELE[_c, ECOL + _c] = 1.0
_SELL = np.zeros((1, WIDE), np.float32)
_SELL[0, LCOL] = 1.0

VB = 4096
NVB = -(-V // VB)

NW = 32
SPW = B // NW
S = 4
NCHUNK = SPW // S
GROWS = S * F


def _sc_body(big, xif, ffm_out, emb_out, xifw_v, g0_v, g1_v, e_v, acc_v,
             sem0, sem1):
  cid = lax.axis_index("c")
  sid = lax.axis_index("s")
  wid = sid * 2 + cid

  pltpu.sync_copy(xif.at[pl.ds(wid * SPW * F, SPW * F)], xifw_v)

  def start(c, g_v, sem):
    pltpu.make_async_copy(
        big.at[xifw_v.at[pl.ds(c * GROWS, GROWS)]], g_v, sem).start()

  def finish_and_compute(c, g_v, sem):
    base_s = wid * SPW + c * S
    pltpu.make_async_copy(
        big.at[xifw_v.at[pl.ds(c * GROWS, GROWS)]], g_v, sem).wait()

    def sample(s, carry2):
      r0 = s * F
      acc = jnp.zeros((16,), jnp.float32)
      for (i, j) in _PAIRS:
        acc = acc + (g_v[r0 + i, pl.ds(16 * j, 16)] *
                     g_v[r0 + j, pl.ds(16 * i, 16)])
      for f in range(F):
        acc = acc + g_v[r0 + f, pl.ds(LCOL, 16)]
        e_v[s, pl.ds(f * D, D)] = g_v[r0 + f, pl.ds(ECOL, D)]
      acc_v[s, :] = acc
      return carry2

    lax.fori_loop(0, S, sample, 0)
    pltpu.sync_copy(acc_v, ffm_out.at[pl.ds(base_s, S)])
    pltpu.sync_copy(e_v, emb_out.at[pl.ds(base_s, S)])

  start(0, g0_v, sem0)

  def pair_of_chunks(nn, carry):
    c0 = 2 * nn
    start(c0 + 1, g1_v, sem1)
    finish_and_compute(c0, g0_v, sem0)

    @pl.when(c0 + 2 < NCHUNK)
    def _():
      start(c0 + 2, g0_v, sem0)

    finish_and_compute(c0 + 1, g1_v, sem1)
    return carry

  lax.fori_loop(0, NCHUNK // 2, pair_of_chunks, 0)


def _sc_gather(big, xif):
  mesh = plsc.VectorSubcoreMesh(core_axis_name="c", subcore_axis_name="s")
  fn = pl.kernel(
      _sc_body,
      out_type=(
          jax.ShapeDtypeStruct((B, 16), jnp.float32),
          jax.ShapeDtypeStruct((B, EOD), jnp.float32),
      ),
      mesh=mesh,
      compiler_params=pltpu.CompilerParams(use_tc_tiling_on_sc=True),
      scratch_types=(
          pltpu.VMEM((SPW * F,), jnp.int32),
          pltpu.VMEM((GROWS, WIDE), jnp.float32),
          pltpu.VMEM((GROWS, WIDE), jnp.float32),
          pltpu.VMEM((S, EOD), jnp.float32),
          pltpu.VMEM((S, 16), jnp.float32),
          pltpu.SemaphoreType.DMA,
          pltpu.SemaphoreType.DMA,
      ),
  )
  return fn(big, xif)


def _pack_body(a3, wembt, wlint, self_, sele, sell, big):
  dn = (((0,), (0,)), ((), ()))
  acc = lax.dot_general(a3[...], self_[...], dn,
                        preferred_element_type=jnp.float32)
  acc = acc + lax.dot_general(wembt[...], sele[...], dn,
                              preferred_element_type=jnp.float32)
  acc = acc + lax.dot_general(wlint[...], sell[...], dn,
                              preferred_element_type=jnp.float32)
  big[...] = acc


def _pack(a3, wembt, wlint):
  return pl.pallas_call(
      _pack_body,
      grid=(NVB,),
      in_specs=[
          pl.BlockSpec((EOD, VB), lambda v: (0, v)),
          pl.BlockSpec((D, VB), lambda v: (0, v)),
          pl.BlockSpec((1, VB), lambda v: (0, v)),
          pl.BlockSpec((EOD, WIDE), lambda v: (0, 0)),
          pl.BlockSpec((D, WIDE), lambda v: (0, 0)),
          pl.BlockSpec((1, WIDE), lambda v: (0, 0)),
      ],
      out_specs=pl.BlockSpec((VB, WIDE), lambda v: (v, 0)),
      out_shape=jax.ShapeDtypeStruct((NVB * VB, WIDE), jnp.float32),
  )(a3, wembt, wlint, jnp.asarray(_SELF), jnp.asarray(_SELE),
    jnp.asarray(_SELL))


def _tc_body(emb, ffmacc, w1, b1, g1, be1, w2, b2, g2, be2, w3, b3,
             blin, out):
  h = emb[...]
  h1 = jnp.dot(h, w1[...], preferred_element_type=jnp.float32) + b1[...]
  mu1 = jnp.mean(h1, axis=0, keepdims=True)
  var1 = jnp.mean((h1 - mu1) ** 2, axis=0, keepdims=True)
  h1 = (h1 - mu1) / jnp.sqrt(var1 + 1e-5) * g1[...] + be1[...]
  h1 = jnp.maximum(h1, 0.0)
  h2 = jnp.dot(h1, w2[...], preferred_element_type=jnp.float32) + b2[...]
  mu2 = jnp.mean(h2, axis=0, keepdims=True)
  var2 = jnp.mean((h2 - mu2) ** 2, axis=0, keepdims=True)
  h2 = (h2 - mu2) / jnp.sqrt(var2 + 1e-5) * g2[...] + be2[...]
  h2 = jnp.maximum(h2, 0.0)
  mlp = jnp.dot(h2, w3[...], preferred_element_type=jnp.float32) + b3[...]
  linffm = jnp.sum(ffmacc[...], axis=1, keepdims=True) + blin[...]
  out[...] = jax.nn.sigmoid(linffm + mlp)


def _tc_head(emb, ffmacc, w1, b1, g1, be1, w2, b2, g2, be2, w3, b3, blin):
  return pl.pallas_call(
      _tc_body,
      out_shape=jax.ShapeDtypeStruct((B, 1), jnp.float32),
  )(emb, ffmacc, w1, b1, g1, be1, w2, b2, g2, be2, w3, b3, blin)


def kernel(x, offsets, W_embed, W_lin, b_lin, W_ffm, W1, b1, g1, be1, W2, b2,
           g2, be2, W3, b3):
  xi = x + offsets[None, :]
  xif = xi.reshape(-1)
  big = _pack(jnp.transpose(W_ffm, (0, 2, 1)).reshape(EOD, V),
              W_embed.T, W_lin.T)

  ffmacc, emb = _sc_gather(big, xif)

  out = _tc_head(
      emb, ffmacc,
      W1, b1.reshape(1, H1), g1.reshape(1, H1), be1.reshape(1, H1),
      W2, b2.reshape(1, H2), g2.reshape(1, H2), be2.reshape(1, H2),
      W3, b3.reshape(1, 1), b_lin.reshape(1, 1))
  return out.reshape(B)
